# Initial kernel scaffold; baseline (speedup 1.0000x reference)
import jax
import jax.numpy as jnp
from jax.experimental import pallas as pl


def kernel(Xb, weight):
    raise NotImplementedError("write your pallas kernel here")


# speedup vs baseline: 1.8755x; 1.8755x over previous
import functools

import jax
import jax.numpy as jnp
from jax import lax
from jax.experimental import pallas as pl
from jax.experimental.pallas import tpu as pltpu
from jax.experimental.pallas import tpu_sc as plsc

D = 64
CHUNK = 128
NBUF = 8


@functools.lru_cache(maxsize=None)
def _make_gather(B):
    info = plsc.get_sparse_core_info()
    NC, NS = info.num_cores, info.num_subcores
    NW = NC * NS
    assert B % (NW * CHUNK) == 0
    b_per_w = B // NW
    n_chunks = b_per_w // CHUNK
    assert n_chunks % NBUF == 0 and n_chunks // NBUF >= 2
    n_outer = n_chunks // NBUF

    mesh = plsc.VectorSubcoreMesh(core_axis_name="c", subcore_axis_name="s")

    scratch = [pltpu.VMEM((n_chunks, CHUNK), jnp.int32)]
    scratch += [pltpu.VMEM((CHUNK, D), jnp.float32) for _ in range(NBUF)]
    scratch += [pltpu.SemaphoreType.DMA for _ in range(2 * NBUF)]

    @functools.partial(
        pl.kernel,
        mesh=mesh,
        out_type=jax.ShapeDtypeStruct((B, D), jnp.float32),
        scratch_types=scratch,
        compiler_params=pltpu.CompilerParams(use_tc_tiling_on_sc=False),
    )
    def gather_kernel(idx_hbm, table_hbm, out_hbm, idx_v, *rest):
        bufs = rest[:NBUF]
        gsems = rest[NBUF:2 * NBUF]
        wsems = rest[2 * NBUF:3 * NBUF]
        wid = lax.axis_index("s") * NC + lax.axis_index("c")
        base = wid * b_per_w

        pltpu.sync_copy(idx_hbm.at[wid], idx_v)

        def start_gather(j, b):
            pltpu.make_async_copy(
                table_hbm.at[idx_v.at[j]], bufs[b], gsems[b]).start()

        def wait_gather(j, b):
            pltpu.make_async_copy(
                table_hbm.at[idx_v.at[j]], bufs[b], gsems[b]).wait()

        def writeback(j, b):
            row0 = base + j * CHUNK
            return pltpu.make_async_copy(
                bufs[b], out_hbm.at[pl.ds(row0, CHUNK)], wsems[b])

        for b in range(NBUF):
            start_gather(b, b)

        def body(g, carry):
            for b in range(NBUF):
                j = g * NBUF + b
                wait_gather(j, b)
                w = writeback(j, b)
                w.start()
                w.wait()
                start_gather(j + NBUF, b)
            return carry

        lax.fori_loop(0, n_outer - 1, body, 0, unroll=False)

        for b in range(NBUF):
            j = (n_outer - 1) * NBUF + b
            wait_gather(j, b)
            writeback(j, b).start()
        for b in range(NBUF):
            j = (n_outer - 1) * NBUF + b
            writeback(j, b).wait()

    def run(idx_flat, table):
        idx3 = idx_flat.reshape(NW, n_chunks, CHUNK)
        return gather_kernel(idx3, table)

    return run


def kernel(Xb, weight):
    B = Xb.shape[0] * Xb.shape[1]
    idx_flat = Xb.reshape(B).astype(jnp.int32)
    out = _make_gather(B)(idx_flat, weight)
    return out.reshape(Xb.shape[0], Xb.shape[1], weight.shape[1])


# speedup vs baseline: 1.9283x; 1.0281x over previous
import functools

import jax
import jax.numpy as jnp
from jax import lax
from jax.experimental import pallas as pl
from jax.experimental.pallas import tpu as pltpu
from jax.experimental.pallas import tpu_sc as plsc

N_ROWS_TBL = 1000000
D = 64
CHUNK = 128
NBUF = 10


def _transpose_in(wt):
    blk = 8192

    def body(x_ref, o_ref):
        o_ref[...] = x_ref[...].T

    return pl.pallas_call(
        body,
        grid=(pl.cdiv(N_ROWS_TBL, blk),),
        in_specs=[pl.BlockSpec((D, blk), lambda g: (0, g))],
        out_specs=pl.BlockSpec((blk, D), lambda g: (g, 0)),
        out_shape=jax.ShapeDtypeStruct((N_ROWS_TBL, D), jnp.float32),
    )(wt)


def _transpose_out(g2, n_b, n_s):
    blk = 4096

    def body(x_ref, o_ref):
        x = x_ref[...]
        o_ref[0] = x[:, :D].T
        o_ref[1] = x[:, D:].T

    return pl.pallas_call(
        body,
        grid=(n_s // 2, n_b // blk),
        in_specs=[pl.BlockSpec((blk, 2 * D), lambda s2, b: (b, s2))],
        out_specs=pl.BlockSpec((2, D, blk), lambda s2, b: (s2, 0, b)),
        out_shape=jax.ShapeDtypeStruct((n_s, D, n_b), jnp.float32),
    )(g2)


@functools.lru_cache(maxsize=None)
def _make_gather(B):
    info = plsc.get_sparse_core_info()
    NC, NS = info.num_cores, info.num_subcores
    NW = NC * NS
    assert B % (NW * CHUNK) == 0
    b_per_w = B // NW
    n_chunks = b_per_w // CHUNK
    assert n_chunks % NBUF == 0 and n_chunks // NBUF >= 2
    n_outer = n_chunks // NBUF

    mesh = plsc.VectorSubcoreMesh(core_axis_name="c", subcore_axis_name="s")

    scratch = [pltpu.VMEM((n_chunks, CHUNK), jnp.int32)]
    scratch += [pltpu.VMEM((CHUNK, D), jnp.float32) for _ in range(NBUF)]
    scratch += [pltpu.SemaphoreType.DMA for _ in range(2 * NBUF)]

    @functools.partial(
        pl.kernel,
        mesh=mesh,
        out_type=jax.ShapeDtypeStruct((B, D), jnp.float32),
        scratch_types=scratch,
        compiler_params=pltpu.CompilerParams(use_tc_tiling_on_sc=False),
    )
    def gather_kernel(idx_hbm, table_hbm, out_hbm, idx_v, *rest):
        bufs = rest[:NBUF]
        gsems = rest[NBUF:2 * NBUF]
        wsems = rest[2 * NBUF:3 * NBUF]
        wid = lax.axis_index("s") * NC + lax.axis_index("c")
        base = wid * b_per_w

        pltpu.sync_copy(idx_hbm.at[wid], idx_v)

        def start_gather(j, b):
            pltpu.make_async_copy(
                table_hbm.at[idx_v.at[j]], bufs[b], gsems[b]).start()

        def wait_gather(j, b):
            pltpu.make_async_copy(
                table_hbm.at[idx_v.at[j]], bufs[b], gsems[b]).wait()

        def writeback(j, b):
            row0 = base + j * CHUNK
            return pltpu.make_async_copy(
                bufs[b], out_hbm.at[pl.ds(row0, CHUNK)], wsems[b])

        for b in range(NBUF):
            start_gather(b, b)

        def body(g, carry):
            for b in range(NBUF):
                j = g * NBUF + b
                wait_gather(j, b)
                w = writeback(j, b)
                w.start()
                w.wait()
                start_gather(j + NBUF, b)
            return carry

        lax.fori_loop(0, n_outer - 1, body, 0, unroll=False)

        for b in range(NBUF):
            j = (n_outer - 1) * NBUF + b
            wait_gather(j, b)
            writeback(j, b).start()
        for b in range(NBUF):
            j = (n_outer - 1) * NBUF + b
            writeback(j, b).wait()

    def run(idx_flat, table):
        idx3 = idx_flat.reshape(NW, n_chunks, CHUNK)
        return gather_kernel(idx3, table)

    return run


def kernel(Xb, weight):
    n_b, n_s = Xb.shape
    B = n_b * n_s
    table = _transpose_in(weight.T)
    idx_flat = Xb.reshape(B).astype(jnp.int32)
    g = _make_gather(B)(idx_flat, table)
    out_t = _transpose_out(g.reshape(n_b, n_s * D), n_b, n_s)
    return jnp.transpose(out_t, (2, 0, 1))


# speedup vs baseline: 3.0496x; 1.5815x over previous
import functools

import jax
import jax.numpy as jnp
from jax import lax
from jax.experimental import pallas as pl
from jax.experimental.pallas import tpu as pltpu
from jax.experimental.pallas import tpu_sc as plsc

N_ROWS_TBL = 1000000
D = 64
CHUNK = 128
NBUF = 5


def _transpose_in(wt):
    blk = 8192

    def body(x_ref, o_ref):
        o_ref[:, :D] = x_ref[...].T

    return pl.pallas_call(
        body,
        grid=(pl.cdiv(N_ROWS_TBL, blk),),
        in_specs=[pl.BlockSpec((D, blk), lambda g: (0, g))],
        out_specs=pl.BlockSpec((blk, 2 * D), lambda g: (g, 0)),
        out_shape=jax.ShapeDtypeStruct((N_ROWS_TBL, 2 * D), jnp.float32),
    )(wt)


def _split_out(g2, n_b, n_s):
    half = n_b // 2
    blk2 = 2048

    def body(x_ref, o_ref):
        j = pl.program_id(1)
        x = x_ref[...]
        o_ref[0, :, pl.ds(j * blk2, blk2)] = x[:, :D].T
        o_ref[0, :, pl.ds(half + j * blk2, blk2)] = x[:, D:].T

    return pl.pallas_call(
        body,
        grid=(n_s, half // blk2),
        in_specs=[pl.BlockSpec(
            (blk2, 2 * D), lambda s, b: (s * (half // blk2) + b, 0))],
        out_specs=pl.BlockSpec((1, D, n_b), lambda s, b: (s, 0, 0)),
        out_shape=jax.ShapeDtypeStruct((n_s, D, n_b), jnp.float32),
    )(g2)


@functools.lru_cache(maxsize=None)
def _make_gather(B, n_b):
    info = plsc.get_sparse_core_info()
    NC, NS = info.num_cores, info.num_subcores
    NW = NC * NS
    assert B % (NW * CHUNK) == 0
    half = n_b // 2
    assert half % CHUNK == 0
    b_per_w = B // NW
    n_chunks = b_per_w // CHUNK
    assert n_chunks % NBUF == 0 and n_chunks // NBUF >= 2
    n_outer = n_chunks // NBUF

    mesh = plsc.VectorSubcoreMesh(core_axis_name="c", subcore_axis_name="s")

    scratch = [pltpu.VMEM((n_chunks, CHUNK), jnp.int32)]
    scratch += [pltpu.VMEM((CHUNK, 2 * D), jnp.float32) for _ in range(NBUF)]
    scratch += [pltpu.SemaphoreType.DMA for _ in range(2 * NBUF)]

    @functools.partial(
        pl.kernel,
        mesh=mesh,
        out_type=jax.ShapeDtypeStruct((B // 2, 2 * D), jnp.float32),
        scratch_types=scratch,
        compiler_params=pltpu.CompilerParams(use_tc_tiling_on_sc=False),
    )
    def gather_kernel(idx_hbm, table_hbm, out_hbm, idx_v, *rest):
        bufs = rest[:NBUF]
        gsems = rest[NBUF:2 * NBUF]
        wsems = rest[2 * NBUF:3 * NBUF]
        wid = lax.axis_index("s") * NC + lax.axis_index("c")
        chunk0 = wid * n_chunks

        pltpu.sync_copy(idx_hbm.at[wid], idx_v)

        def start_gather(j, b):
            pltpu.make_async_copy(
                table_hbm.at[idx_v.at[j]], bufs[b], gsems[b]).start()

        def wait_gather(j, b):
            pltpu.make_async_copy(
                table_hbm.at[idx_v.at[j]], bufs[b], gsems[b]).wait()

        def writeback(j, b):
            r0 = (chunk0 + j) * CHUNK
            s = r0 // n_b
            b0 = r0 % n_b
            u0 = s * half + (b0 % half)
            col0 = (b0 // half) * D
            return pltpu.make_async_copy(
                bufs[b].at[:, :D],
                out_hbm.at[pl.ds(u0, CHUNK), pl.ds(col0, D)],
                wsems[b])

        for b in range(NBUF):
            start_gather(b, b)

        def body(g, carry):
            for b in range(NBUF):
                j = g * NBUF + b
                wait_gather(j, b)
                w = writeback(j, b)
                w.start()
                w.wait()
                start_gather(j + NBUF, b)
            return carry

        lax.fori_loop(0, n_outer - 1, body, 0, unroll=False)

        for b in range(NBUF):
            j = (n_outer - 1) * NBUF + b
            wait_gather(j, b)
            writeback(j, b).start()
        for b in range(NBUF):
            j = (n_outer - 1) * NBUF + b
            writeback(j, b).wait()

    def run(idx_flat, table):
        idx3 = idx_flat.reshape(NW, n_chunks, CHUNK)
        return gather_kernel(idx3, table)

    return run


def kernel(Xb, weight):
    n_b, n_s = Xb.shape
    B = n_b * n_s
    table = _transpose_in(weight.T)
    idx_t = Xb.T.reshape(B).astype(jnp.int32)
    g2 = _make_gather(B, n_b)(idx_t, table)
    out_t = _split_out(g2, n_b, n_s)
    return jnp.transpose(out_t, (2, 0, 1))


# speedup vs baseline: 3.4583x; 1.1340x over previous
import functools

import jax
import jax.numpy as jnp
from jax import lax
from jax.experimental import pallas as pl
from jax.experimental.pallas import tpu as pltpu
from jax.experimental.pallas import tpu_sc as plsc

N_ROWS_TBL = 1000000
D = 64
CHUNK = 128
NBUF = 5


def _transpose_in(wt):
    blk = 8192

    def body(x_ref, o_ref):
        o_ref[:, :D] = x_ref[...].T

    return pl.pallas_call(
        body,
        grid=(pl.cdiv(N_ROWS_TBL, blk),),
        in_specs=[pl.BlockSpec((D, blk), lambda g: (0, g))],
        out_specs=pl.BlockSpec((blk, 2 * D), lambda g: (g, 0)),
        out_shape=jax.ShapeDtypeStruct((N_ROWS_TBL, 2 * D), jnp.float32),
    )(wt)


def _split_out(g2, n_b, n_s):
    half = n_b // 2
    blk2 = 8192

    def body(x_ref, o_ref):
        j = pl.program_id(1)
        x = x_ref[...]
        o_ref[0, :, pl.ds(j * blk2, blk2)] = x[:, :D].T
        o_ref[0, :, pl.ds(half + j * blk2, blk2)] = x[:, D:].T

    return pl.pallas_call(
        body,
        grid=(n_s, half // blk2),
        in_specs=[pl.BlockSpec(
            (blk2, 2 * D), lambda s, b: (s * (half // blk2) + b, 0))],
        out_specs=pl.BlockSpec((1, D, n_b), lambda s, b: (s, 0, 0)),
        out_shape=jax.ShapeDtypeStruct((n_s, D, n_b), jnp.float32),
    )(g2)


@functools.lru_cache(maxsize=None)
def _make_gather(B, n_b):
    info = plsc.get_sparse_core_info()
    NC, NS = info.num_cores, info.num_subcores
    NW = NC * NS
    assert B % (NW * CHUNK) == 0
    half = n_b // 2
    assert half % CHUNK == 0
    b_per_w = B // NW
    n_chunks = b_per_w // CHUNK
    assert n_chunks % NBUF == 0 and n_chunks // NBUF >= 2
    n_outer = n_chunks // NBUF

    mesh = plsc.VectorSubcoreMesh(core_axis_name="c", subcore_axis_name="s")

    scratch = [pltpu.VMEM((n_chunks, CHUNK), jnp.int32)]
    scratch += [pltpu.VMEM((CHUNK, 2 * D), jnp.float32) for _ in range(NBUF)]
    scratch += [pltpu.SemaphoreType.DMA for _ in range(2 * NBUF)]

    @functools.partial(
        pl.kernel,
        mesh=mesh,
        out_type=jax.ShapeDtypeStruct((B // 2, 2 * D), jnp.float32),
        scratch_types=scratch,
        compiler_params=pltpu.CompilerParams(use_tc_tiling_on_sc=False),
    )
    def gather_kernel(idx_hbm, table_hbm, out_hbm, idx_v, *rest):
        bufs = rest[:NBUF]
        gsems = rest[NBUF:2 * NBUF]
        wsems = rest[2 * NBUF:3 * NBUF]
        wid = lax.axis_index("s") * NC + lax.axis_index("c")
        chunk0 = wid * n_chunks

        pltpu.sync_copy(idx_hbm.at[wid], idx_v)

        def start_gather(j, b):
            pltpu.make_async_copy(
                table_hbm.at[idx_v.at[j]], bufs[b], gsems[b]).start()

        def wait_gather(j, b):
            pltpu.make_async_copy(
                table_hbm.at[idx_v.at[j]], bufs[b], gsems[b]).wait()

        def writeback(j, b):
            r0 = (chunk0 + j) * CHUNK
            s = r0 // n_b
            b0 = r0 % n_b
            u0 = s * half + (b0 % half)
            col0 = (b0 // half) * D
            return pltpu.make_async_copy(
                bufs[b].at[:, :D],
                out_hbm.at[pl.ds(u0, CHUNK), pl.ds(col0, D)],
                wsems[b])

        for b in range(NBUF):
            start_gather(b, b)

        def body(g, carry):
            for b in range(NBUF):
                j = g * NBUF + b
                wait_gather(j, b)
                w = writeback(j, b)
                w.start()
                w.wait()
                start_gather(j + NBUF, b)
            return carry

        lax.fori_loop(0, n_outer - 1, body, 0, unroll=False)

        for b in range(NBUF):
            j = (n_outer - 1) * NBUF + b
            wait_gather(j, b)
            writeback(j, b).start()
        for b in range(NBUF):
            j = (n_outer - 1) * NBUF + b
            writeback(j, b).wait()

    def run(idx_flat, table):
        idx3 = idx_flat.reshape(NW, n_chunks, CHUNK)
        return gather_kernel(idx3, table)

    return run


def kernel(Xb, weight):
    n_b, n_s = Xb.shape
    B = n_b * n_s
    table = _transpose_in(weight.T)
    idx_t = Xb.T.reshape(B).astype(jnp.int32)
    g2 = _make_gather(B, n_b)(idx_t, table)
    out_t = _split_out(g2, n_b, n_s)
    return jnp.transpose(out_t, (2, 0, 1))


# speedup vs baseline: 3.5198x; 1.0178x over previous
import functools

import jax
import jax.numpy as jnp
from jax import lax
from jax.experimental import pallas as pl
from jax.experimental.pallas import tpu as pltpu
from jax.experimental.pallas import tpu_sc as plsc

N_ROWS_TBL = 1000000
D = 64
CHUNK = 128
NBUF = 5
N_WAVES = 2


def _transpose_in(wt):
    blk = 8192

    def body(x_ref, o_ref):
        o_ref[:, :D] = x_ref[...].T

    return pl.pallas_call(
        body,
        grid=(pl.cdiv(N_ROWS_TBL, blk),),
        in_specs=[pl.BlockSpec((D, blk), lambda g: (0, g))],
        out_specs=pl.BlockSpec((blk, 2 * D), lambda g: (g, 0)),
        out_shape=jax.ShapeDtypeStruct((N_ROWS_TBL, 2 * D), jnp.float32),
    )(wt)


def _split_out_wave(g2w, prev, wave, n_b, n_s):
    half = n_b // 2
    n_sw = n_s // N_WAVES
    s_off = wave * n_sw

    def body(x_ref, *refs):
        o_ref = refs[-1]
        x = x_ref[...]
        o_ref[0, :, pl.ds(0, half)] = x[:, :D].T
        o_ref[0, :, pl.ds(half, half)] = x[:, D:].T

    out_spec = pl.BlockSpec((1, D, n_b), lambda s: (s_off + s, 0, 0))
    in_specs = [pl.BlockSpec((half, 2 * D), lambda s: (s, 0))]
    operands = [g2w]
    kwargs = {}
    if prev is not None:
        in_specs.append(pl.BlockSpec(memory_space=pl.ANY))
        operands.append(prev)
        kwargs["input_output_aliases"] = {1: 0}

    return pl.pallas_call(
        body,
        grid=(n_sw,),
        in_specs=in_specs,
        out_specs=out_spec,
        out_shape=jax.ShapeDtypeStruct((n_s, D, n_b), jnp.float32),
        **kwargs,
    )(*operands)


@functools.lru_cache(maxsize=None)
def _make_gather(B, n_b, wave):
    info = plsc.get_sparse_core_info()
    NC, NS = info.num_cores, info.num_subcores
    NW = NC * NS
    Bw = B // N_WAVES
    assert Bw % (NW * CHUNK) == 0
    half = n_b // 2
    assert half % CHUNK == 0
    n_chunks = Bw // NW // CHUNK
    assert n_chunks % NBUF == 0 and n_chunks // NBUF >= 2
    n_outer = n_chunks // NBUF
    wave_chunk0 = wave * (Bw // CHUNK)

    mesh = plsc.VectorSubcoreMesh(core_axis_name="c", subcore_axis_name="s")

    scratch = [pltpu.VMEM((n_chunks, CHUNK), jnp.int32)]
    scratch += [pltpu.VMEM((CHUNK, 2 * D), jnp.float32) for _ in range(NBUF)]
    scratch += [pltpu.SemaphoreType.DMA for _ in range(2 * NBUF)]

    @functools.partial(
        pl.kernel,
        mesh=mesh,
        out_type=jax.ShapeDtypeStruct((Bw // 2, 2 * D), jnp.float32),
        scratch_types=scratch,
        compiler_params=pltpu.CompilerParams(use_tc_tiling_on_sc=False),
    )
    def gather_kernel(idx_hbm, table_hbm, out_hbm, idx_v, *rest):
        bufs = rest[:NBUF]
        gsems = rest[NBUF:2 * NBUF]
        wsems = rest[2 * NBUF:3 * NBUF]
        wid = lax.axis_index("s") * NC + lax.axis_index("c")
        chunk0 = wid * n_chunks

        pltpu.sync_copy(idx_hbm.at[wid], idx_v)

        def start_gather(j, b):
            pltpu.make_async_copy(
                table_hbm.at[idx_v.at[j]], bufs[b], gsems[b]).start()

        def wait_gather(j, b):
            pltpu.make_async_copy(
                table_hbm.at[idx_v.at[j]], bufs[b], gsems[b]).wait()

        def writeback(j, b):
            r0 = (wave_chunk0 + chunk0 + j) * CHUNK
            s = r0 // n_b
            b0 = r0 % n_b
            u0 = (s - wave * (Bw // n_b)) * half + (b0 % half)
            col0 = (b0 // half) * D
            return pltpu.make_async_copy(
                bufs[b].at[:, :D],
                out_hbm.at[pl.ds(u0, CHUNK), pl.ds(col0, D)],
                wsems[b])

        for b in range(NBUF):
            start_gather(b, b)

        def body(g, carry):
            for b in range(NBUF):
                j = g * NBUF + b
                wait_gather(j, b)
                w = writeback(j, b)
                w.start()
                w.wait()
                start_gather(j + NBUF, b)
            return carry

        lax.fori_loop(0, n_outer - 1, body, 0, unroll=False)

        for b in range(NBUF):
            j = (n_outer - 1) * NBUF + b
            wait_gather(j, b)
            writeback(j, b).start()
        for b in range(NBUF):
            j = (n_outer - 1) * NBUF + b
            writeback(j, b).wait()

    def run(idx_wave, table):
        idx3 = idx_wave.reshape(NW, n_chunks, CHUNK)
        return gather_kernel(idx3, table)

    return run


def kernel(Xb, weight):
    n_b, n_s = Xb.shape
    B = n_b * n_s
    Bw = B // N_WAVES
    table = _transpose_in(weight.T)
    idx_t = Xb.T.reshape(B).astype(jnp.int32)
    g2 = [
        _make_gather(B, n_b, w)(idx_t[w * Bw:(w + 1) * Bw], table)
        for w in range(N_WAVES)
    ]
    out = None
    for w in range(N_WAVES):
        out = _split_out_wave(g2[w], out, w, n_b, n_s)
    return jnp.transpose(out, (2, 0, 1))


# speedup vs baseline: 3.6530x; 1.0378x over previous
import functools

import jax
import jax.numpy as jnp
from jax import lax
from jax.experimental import pallas as pl
from jax.experimental.pallas import tpu as pltpu
from jax.experimental.pallas import tpu_sc as plsc

N_ROWS_TBL = 1000000
D = 64
CHUNK = 128
NBUF = 5
N_WAVES = 5


def _transpose_in(wt):
    blk = 16384

    def body(x_ref, o_ref):
        o_ref[:, :D] = x_ref[...].T

    return pl.pallas_call(
        body,
        grid=(pl.cdiv(N_ROWS_TBL, blk),),
        in_specs=[pl.BlockSpec((D, blk), lambda g: (0, g))],
        out_specs=pl.BlockSpec((blk, 2 * D), lambda g: (g, 0)),
        out_shape=jax.ShapeDtypeStruct((N_ROWS_TBL, 2 * D), jnp.float32),
    )(wt)


def _split_out_wave(g2w, prev, wave, n_b, n_s):
    half = n_b // 2
    n_sw = n_s // N_WAVES
    s_off = wave * n_sw

    def body(x_ref, *refs):
        o_ref = refs[-1]
        x = x_ref[...]
        o_ref[0, :, pl.ds(0, half)] = x[:, :D].T
        o_ref[0, :, pl.ds(half, half)] = x[:, D:].T

    out_spec = pl.BlockSpec((1, D, n_b), lambda s: (s_off + s, 0, 0))
    in_specs = [pl.BlockSpec((half, 2 * D), lambda s: (s, 0))]
    operands = [g2w]
    kwargs = {}
    if prev is not None:
        in_specs.append(pl.BlockSpec(memory_space=pl.ANY))
        operands.append(prev)
        kwargs["input_output_aliases"] = {1: 0}

    return pl.pallas_call(
        body,
        grid=(n_sw,),
        in_specs=in_specs,
        out_specs=out_spec,
        out_shape=jax.ShapeDtypeStruct((n_s, D, n_b), jnp.float32),
        **kwargs,
    )(*operands)


@functools.lru_cache(maxsize=None)
def _make_gather(B, n_b, wave):
    info = plsc.get_sparse_core_info()
    NC, NS = info.num_cores, info.num_subcores
    NW = NC * NS
    Bw = B // N_WAVES
    assert Bw % (NW * CHUNK) == 0
    half = n_b // 2
    assert half % CHUNK == 0
    n_chunks = Bw // NW // CHUNK
    assert n_chunks % NBUF == 0 and n_chunks // NBUF >= 2
    n_outer = n_chunks // NBUF
    wave_chunk0 = wave * (Bw // CHUNK)

    mesh = plsc.VectorSubcoreMesh(core_axis_name="c", subcore_axis_name="s")

    scratch = [pltpu.VMEM((n_chunks, CHUNK), jnp.int32)]
    scratch += [pltpu.VMEM((CHUNK, 2 * D), jnp.float32) for _ in range(NBUF)]
    scratch += [pltpu.SemaphoreType.DMA for _ in range(2 * NBUF)]

    @functools.partial(
        pl.kernel,
        mesh=mesh,
        out_type=jax.ShapeDtypeStruct((Bw // 2, 2 * D), jnp.float32),
        scratch_types=scratch,
        compiler_params=pltpu.CompilerParams(use_tc_tiling_on_sc=False),
    )
    def gather_kernel(idx_hbm, table_hbm, out_hbm, idx_v, *rest):
        bufs = rest[:NBUF]
        gsems = rest[NBUF:2 * NBUF]
        wsems = rest[2 * NBUF:3 * NBUF]
        wid = lax.axis_index("s") * NC + lax.axis_index("c")
        chunk0 = wid * n_chunks

        pltpu.sync_copy(idx_hbm.at[wid], idx_v)

        def start_gather(j, b):
            pltpu.make_async_copy(
                table_hbm.at[idx_v.at[j]], bufs[b], gsems[b]).start()

        def wait_gather(j, b):
            pltpu.make_async_copy(
                table_hbm.at[idx_v.at[j]], bufs[b], gsems[b]).wait()

        def writeback(j, b):
            r0 = (wave_chunk0 + chunk0 + j) * CHUNK
            s = r0 // n_b
            b0 = r0 % n_b
            u0 = (s - wave * (Bw // n_b)) * half + (b0 % half)
            col0 = (b0 // half) * D
            return pltpu.make_async_copy(
                bufs[b].at[:, :D],
                out_hbm.at[pl.ds(u0, CHUNK), pl.ds(col0, D)],
                wsems[b])

        for b in range(NBUF):
            start_gather(b, b)

        def body(g, carry):
            for b in range(NBUF):
                j = g * NBUF + b
                wait_gather(j, b)
                w = writeback(j, b)
                w.start()
                w.wait()
                start_gather(j + NBUF, b)
            return carry

        lax.fori_loop(0, n_outer - 1, body, 0, unroll=False)

        for b in range(NBUF):
            j = (n_outer - 1) * NBUF + b
            wait_gather(j, b)
            writeback(j, b).start()
        for b in range(NBUF):
            j = (n_outer - 1) * NBUF + b
            writeback(j, b).wait()

    def run(idx_wave, table):
        idx3 = idx_wave.reshape(NW, n_chunks, CHUNK)
        return gather_kernel(idx3, table)

    return run


def kernel(Xb, weight):
    n_b, n_s = Xb.shape
    B = n_b * n_s
    Bw = B // N_WAVES
    table = _transpose_in(weight.T)
    idx_t = Xb.T.reshape(B).astype(jnp.int32)
    g2 = [
        _make_gather(B, n_b, w)(idx_t[w * Bw:(w + 1) * Bw], table)
        for w in range(N_WAVES)
    ]
    out = None
    for w in range(N_WAVES):
        out = _split_out_wave(g2[w], out, w, n_b, n_s)
    return jnp.transpose(out, (2, 0, 1))
